# baseline (speedup 1.0000x reference)
import functools

import jax
import jax.numpy as jnp
from jax import lax
from jax.experimental import pallas as pl
from jax.experimental.pallas import tpu as pltpu
from jax.experimental.pallas import tpu_sc as plsc

_HEADS = 9
_NEG = -1e30
_BN = 1024


def _prep_call(xs, Ws, Us, Qs, pool):
    NP = xs[0].shape[0]
    nx = len(xs)
    D9 = Ws[0].shape[1]
    DU = Us[0].shape[1]
    cout = Qs[0].shape[1]
    grid = pl.cdiv(NP, _BN)

    nw = 1 if pool else nx

    def body(*refs):
        x_refs = refs[:nx]
        w_refs = refs[nx:nx + nw]
        u_refs = refs[nx + nw:nx + 2 * nw]
        q_refs = refs[nx + 2 * nw:nx + 3 * nw]
        xw_ref, u_ref, s_ref = refs[nx + 3 * nw:]
        if pool:
            xin = jnp.maximum(x_refs[0][...], x_refs[1][...])
            xw = jnp.dot(xin, w_refs[0][...], preferred_element_type=jnp.float32)
            uu = jnp.dot(xin, u_refs[0][...], preferred_element_type=jnp.float32)
            ss = jnp.dot(xin, q_refs[0][...], preferred_element_type=jnp.float32)
        else:
            xw = jnp.dot(x_refs[0][...], w_refs[0][...], preferred_element_type=jnp.float32)
            uu = jnp.dot(x_refs[0][...], u_refs[0][...], preferred_element_type=jnp.float32)
            ss = jnp.dot(x_refs[0][...], q_refs[0][...], preferred_element_type=jnp.float32)
            for i in range(1, nx):
                xw += jnp.dot(x_refs[i][...], w_refs[i][...], preferred_element_type=jnp.float32)
                uu += jnp.dot(x_refs[i][...], u_refs[i][...], preferred_element_type=jnp.float32)
                ss += jnp.dot(x_refs[i][...], q_refs[i][...], preferred_element_type=jnp.float32)
        xw_ref[...] = xw
        u_ref[...] = uu
        s_ref[...] = ss

    in_specs = (
        [pl.BlockSpec((_BN, xs[i].shape[1]), lambda i_: (i_, 0)) for i in range(nx)]
        + [pl.BlockSpec(Ws[i].shape, lambda i_: (0, 0)) for i in range(nw)]
        + [pl.BlockSpec(Us[i].shape, lambda i_: (0, 0)) for i in range(nw)]
        + [pl.BlockSpec(Qs[i].shape, lambda i_: (0, 0)) for i in range(nw)]
    )
    out_specs = [
        pl.BlockSpec((_BN, D9), lambda i_: (i_, 0)),
        pl.BlockSpec((_BN, DU), lambda i_: (i_, 0)),
        pl.BlockSpec((_BN, cout), lambda i_: (i_, 0)),
    ]
    out_shape = [
        jax.ShapeDtypeStruct((NP, D9), jnp.float32),
        jax.ShapeDtypeStruct((NP, DU), jnp.float32),
        jax.ShapeDtypeStruct((NP, cout), jnp.float32),
    ]
    return pl.pallas_call(
        body, grid=(grid,), in_specs=in_specs, out_specs=out_specs,
        out_shape=out_shape,
    )(*xs, *Ws[:nw], *Us[:nw], *Qs[:nw])


def _head_call(feat, w1, b1, w2, b2, xinit):
    NP = feat.shape[0]
    grid = pl.cdiv(NP, _BN)

    def body(f_ref, w1_ref, b1_ref, w2_ref, b2_ref, xi_ref, o_ref):
        h = jnp.dot(f_ref[...], w1_ref[...], preferred_element_type=jnp.float32)
        h = h + b1_ref[...]
        h = jnp.where(h > 0, h, 0.2 * h)
        o = jnp.dot(h, w2_ref[...], preferred_element_type=jnp.float32)
        o = o + b2_ref[...] + xi_ref[...]
        nrm = jnp.sqrt(jnp.sum(o * o, axis=1, keepdims=True))
        o_ref[...] = o / jnp.maximum(nrm, 1e-12)

    return pl.pallas_call(
        body,
        grid=(grid,),
        in_specs=[
            pl.BlockSpec((_BN, feat.shape[1]), lambda i: (i, 0)),
            pl.BlockSpec(w1.shape, lambda i: (0, 0)),
            pl.BlockSpec(b1.shape, lambda i: (0, 0)),
            pl.BlockSpec(w2.shape, lambda i: (0, 0)),
            pl.BlockSpec(b2.shape, lambda i: (0, 0)),
            pl.BlockSpec((_BN, 3), lambda i: (i, 0)),
        ],
        out_specs=pl.BlockSpec((_BN, 3), lambda i: (i, 0)),
        out_shape=jax.ShapeDtypeStruct((NP, 3), jnp.float32),
    )(feat, w1, b1, w2, b2, xinit)


@functools.lru_cache(maxsize=None)
def _sc_conv(cout, shift_g, shift_s, NPs, with_deg, EPT):
    W = -(-(cout + (16 if with_deg else 0)) // 128) * 128
    UOFF = _HEADS * cout
    D9 = -(-(UOFF + _HEADS) // 128) * 128
    NB = EPT // 16
    DEPTH = 2
    MBD = 2
    NK = cout // 16
    rps = NPs // 16
    mesh = plsc.VectorSubcoreMesh(core_axis_name="c", subcore_axis_name="s",
                                  num_cores=2, num_subcores=16)

    def body(xw_hbm, u_hbm, cvec_hbm, src_hbm, dst_hbm, zeros_hbm, out_hbm,
             src_v, dst_v, du_b, gx_b, mb_b, cv_v, agg_sp, du_s, gx_s, mb_s):
        cid = lax.axis_index("c")
        sid = lax.axis_index("s")
        wid = sid * 2 + cid
        pltpu.sync_copy(zeros_hbm.at[pl.ds(sid * rps, rps)],
                        agg_sp.at[pl.ds(sid * rps, rps)])
        pltpu.sync_copy(cvec_hbm, cv_v)
        plsc.subcore_barrier()
        cvec = cv_v[...]
        ii = lax.iota(jnp.int32, 16)
        degv = jnp.where(ii == 0, jnp.float32(1.0 if with_deg else 0.0),
                         jnp.float32(0.0))
        zv = jnp.zeros((16,), jnp.float32)
        for j in range(MBD):
            for e in range(16):
                for col in range(cout, W, 16):
                    mb_b[j][e, pl.ds(col, 16)] = degv if col == cout else zv

        sets = tuple((du_b[j], gx_b[j], du_s[j], gx_s[j],
                      mb_b[j % MBD], mb_s[j % MBD]) for j in range(DEPTH))

        def mk_idx(b):
            s_raw = src_v[pl.ds(b * 16, 16)]
            d_raw = dst_v[pl.ds(b * 16, 16)]
            sg = lax.shift_right_logical(s_raw, shift_g) if shift_g else s_raw
            dg = lax.shift_right_logical(d_raw, shift_g) if shift_g else d_raw
            dsc = lax.shift_right_logical(d_raw, shift_s) if shift_s else d_raw
            return sg, dg, dsc

        def issue(b, st):
            du, gx, dus, gxs = st[:4]
            sg, dg, _ = mk_idx(b)
            pltpu.async_copy(u_hbm.at[dg], du, dus)
            pltpu.async_copy(xw_hbm.at[sg], gx, gxs)

        def compute(b, st):
            du, gx, dus, gxs, mb, mbs = st
            sg, dg, dsc = mk_idx(b)
            pltpu.make_async_copy(u_hbm.at[dg], du, dus).wait()
            pltpu.make_async_copy(xw_hbm.at[sg], gx, gxs).wait()

            @pl.when(b >= MBD)
            def _():
                pltpu.make_async_copy(mb, agg_sp.at[dsc], mbs).wait()

            def edge_body(e, c2):
                t = gx[e, pl.ds(UOFF, 16)] - du[e, pl.ds(0, 16)] + cvec
                p = jnp.exp(t)
                ph = [p[h] for h in range(_HEADS)]
                ssum = ph[0]
                for h in range(1, _HEADS):
                    ssum = ssum + ph[h]
                invv = 1.0 / jnp.broadcast_to(ssum, (16,))
                accs = [jnp.zeros((16,), jnp.float32)] * NK
                for h in range(_HEADS):
                    for k in range(NK):
                        accs[k] = accs[k] + ph[h] * gx[e, pl.ds(h * cout + k * 16, 16)]
                for k in range(NK):
                    mb[e, pl.ds(k * 16, 16)] = accs[k] * invv
                return c2

            lax.fori_loop(0, 16, edge_body, 0)
            pltpu.async_copy(mb, agg_sp.at[dsc], mbs, add=True)

        zi = jnp.zeros((16,), jnp.int32)
        pltpu.sync_copy(src_hbm.at[pl.ds(wid * EPT, EPT)], src_v)
        pltpu.sync_copy(dst_hbm.at[pl.ds(wid * EPT, EPT)], dst_v)
        for j in range(DEPTH - 1):
            issue(j, sets[j])

        def quad_body(k, carry):
            b0 = DEPTH * k
            for j in range(DEPTH):
                b = b0 + j

                @pl.when(b + DEPTH - 1 < NB)
                def _():
                    issue(b + DEPTH - 1, sets[(j + DEPTH - 1) % DEPTH])

                compute(b, sets[j])
            return carry

        lax.fori_loop(0, NB // DEPTH, quad_body, 0)
        for j in range(MBD):
            pltpu.make_async_copy(mb_b[j], agg_sp.at[zi], mb_s[j]).wait()
        plsc.subcore_barrier()
        pltpu.sync_copy(agg_sp.at[pl.ds(sid * rps, rps)],
                        out_hbm.at[cid, pl.ds(sid * rps, rps)])

    return pl.kernel(
        body,
        out_type=jax.ShapeDtypeStruct((2, NPs, W), jnp.float32),
        mesh=mesh,
        scratch_types=[
            pltpu.VMEM((EPT,), jnp.int32),
            pltpu.VMEM((EPT,), jnp.int32),
            [pltpu.VMEM((16, 128), jnp.float32)] * DEPTH,
            [pltpu.VMEM((16, D9), jnp.float32)] * DEPTH,
            [pltpu.VMEM((16, W), jnp.float32)] * MBD,
            pltpu.VMEM((16,), jnp.float32),
            pltpu.VMEM_SHARED((NPs, W), jnp.float32),
            [pltpu.SemaphoreType.DMA] * DEPTH,
            [pltpu.SemaphoreType.DMA] * DEPTH,
            [pltpu.SemaphoreType.DMA] * MBD,
        ],
    )


def _split_params(p, splits):
    W, u, c, b = p["W"], p["u"], p["c"], p["b"]
    cout = b.shape[0]
    qc = jax.nn.softmax(c)
    wqc = (W.reshape(-1, _HEADS, cout) * qc[None, :, None]).sum(1)
    D9 = W.shape[1]
    D9p = -(-(D9 + _HEADS) // 128) * 128
    W = jnp.pad(jnp.concatenate([W, u], axis=1), ((0, 0), (0, D9p - D9 - _HEADS)))
    u16 = jnp.pad(u, ((0, 0), (0, 128 - _HEADS)))
    cvec = jnp.concatenate([c, jnp.full((16 - _HEADS,), _NEG, jnp.float32)])
    Ws, Us, Qs = [], [], []
    o = 0
    for s in splits:
        Ws.append(W[o:o + s])
        Us.append(u16[o:o + s])
        Qs.append(wqc[o:o + s])
        o += s
    return Ws, Us, Qs, cvec, b.reshape(1, cout)


def _lrelu(v):
    return jnp.where(v > 0, v, 0.2 * v)


def kernel(x, edge_index, params):
    n1 = x.shape[0]
    n2 = (n1 + 1) // 2
    n3 = (n2 + 1) // 2
    NP1, NP2, NP3 = [-(-(n + 1) // 128) * 128 for n in (n1, n2, n3)]
    E = edge_index.shape[1]
    EPT = -(-(-(-E // 32)) // 64) * 64
    EP = 32 * EPT
    src = jnp.concatenate([edge_index[0], jnp.full((EP - E,), n1, jnp.int32)])
    dst = jnp.concatenate([edge_index[1], jnp.full((EP - E,), n1, jnp.int32)])
    f32 = jnp.float32

    def conv(xs, pname, splits, pool, shift_g, shift_s, NPg, NPs, with_deg=False):
        Ws, Us, Qs, cvec, b = _split_params(params[pname], splits)
        cout = b.shape[1]
        xw, u16t, s = _prep_call(xs, Ws, Us, Qs, pool)
        Wagg = -(-(cout + (16 if with_deg else 0)) // 128) * 128
        zeros = jnp.zeros((NPs, Wagg), f32)
        part = _sc_conv(cout, shift_g, shift_s, NPs, with_deg, EPT)(
            xw, u16t, cvec, src, dst, zeros)
        return part, s, b

    xp = jnp.pad(x, ((0, NP1 - n1), (0, 0)))
    p1, s1, b1 = conv([xp], "l1", [6], False, 0, 0, NP1, NP1, with_deg=True)
    indeg1 = p1[0, :, 32] + p1[1, :, 32]
    deg1 = indeg1 + 1.0
    deg2 = jnp.pad(indeg1.reshape(NP1 // 2, 2).sum(1),
                   (0, NP2 - NP1 // 2)) + 1.0
    deg3 = jnp.pad((deg2 - 1.0)[:NP2].reshape(NP2 // 2, 2).sum(1),
                   (0, NP3 - NP2 // 2)) + 1.0
    y1 = _lrelu((p1[0, :, :32] + p1[1, :, :32] + s1) / deg1[:, None] + b1)

    pad2 = ((0, NP2 - NP1 // 2), (0, 0))
    p2, s2, b2 = conv([jnp.pad(y1[0::2], pad2), jnp.pad(y1[1::2], pad2)],
                      "l2", [32], True, 1, 1, NP2, NP2)
    y2 = _lrelu((p2[0, :, :64] + p2[1, :, :64] + s2) / deg2[:, None] + b2)

    pad3 = ((0, NP3 - NP2 // 2), (0, 0))
    p3, s3, b3 = conv([jnp.pad(y2[0::2], pad3), jnp.pad(y2[1::2], pad3)],
                      "l3", [64], True, 2, 2, NP3, NP3)
    y3a = _lrelu((p3[0, :, :128] + p3[1, :, :128] + s3) / deg3[:, None] + b3)

    p4, s4, b4 = conv([y3a], "l4", [128], False, 2, 2, NP3, NP3)
    y3 = _lrelu((p4[0, :, :128] + p4[1, :, :128] + s4) / deg3[:, None] + b4)

    pr1, sr1, br1 = conv([y3], "r1", [128], False, 2, 1, NP3, NP2)
    sr1 = jnp.repeat(sr1[:NP2 // 2], 2, axis=0)
    f2p = (pr1[0, :, :64] + pr1[1, :, :64] + sr1) / deg2[:, None] + br1

    pr2, sr2, br2 = conv([y2, f2p], "r2", [64, 64], False, 1, 1, NP2, NP2)
    y2c = _lrelu((pr2[0, :, :64] + pr2[1, :, :64] + sr2) / deg2[:, None] + br2)

    pr3, sr3, br3 = conv([y2c], "r3", [64], False, 1, 0, NP2, NP1)
    sr3 = jnp.repeat(sr3[:NP1 // 2], 2, axis=0)
    f1p = (pr3[0, :, :32] + pr3[1, :, :32] + sr3) / deg1[:, None] + br3

    pr4, sr4, br4 = conv([y1, f1p], "r4", [32, 32], False, 0, 0, NP1, NP1)
    feat = _lrelu((pr4[0, :, :32] + pr4[1, :, :32] + sr4) / deg1[:, None] + br4)

    xinit = jnp.pad(x[:, 3:6], ((0, NP1 - n1), (0, 0)))
    out = _head_call(feat, params["fc_v1"]["W"], params["fc_v1"]["b"].reshape(1, -1),
                     params["fc_v2"]["W"], params["fc_v2"]["b"].reshape(1, -1), xinit)
    return out[:n1]


# speedup vs baseline: 1.0002x; 1.0002x over previous
import functools

import jax
import jax.numpy as jnp
from jax import lax
from jax.experimental import pallas as pl
from jax.experimental.pallas import tpu as pltpu
from jax.experimental.pallas import tpu_sc as plsc

_HEADS = 9
_NEG = -1e30
_BN = 1024


def _prep_call(xs, Ws, Us, Qs, pool):
    NP = xs[0].shape[0]
    nx = len(xs)
    D9 = Ws[0].shape[1]
    DU = Us[0].shape[1]
    cout = Qs[0].shape[1]
    grid = pl.cdiv(NP, _BN)

    nw = 1 if pool else nx

    def body(*refs):
        x_refs = refs[:nx]
        w_refs = refs[nx:nx + nw]
        u_refs = refs[nx + nw:nx + 2 * nw]
        q_refs = refs[nx + 2 * nw:nx + 3 * nw]
        xw_ref, u_ref, s_ref = refs[nx + 3 * nw:]
        if pool:
            xin = jnp.maximum(x_refs[0][...], x_refs[1][...])
            xw = jnp.dot(xin, w_refs[0][...], preferred_element_type=jnp.float32)
            uu = jnp.dot(xin, u_refs[0][...], preferred_element_type=jnp.float32)
            ss = jnp.dot(xin, q_refs[0][...], preferred_element_type=jnp.float32)
        else:
            xw = jnp.dot(x_refs[0][...], w_refs[0][...], preferred_element_type=jnp.float32)
            uu = jnp.dot(x_refs[0][...], u_refs[0][...], preferred_element_type=jnp.float32)
            ss = jnp.dot(x_refs[0][...], q_refs[0][...], preferred_element_type=jnp.float32)
            for i in range(1, nx):
                xw += jnp.dot(x_refs[i][...], w_refs[i][...], preferred_element_type=jnp.float32)
                uu += jnp.dot(x_refs[i][...], u_refs[i][...], preferred_element_type=jnp.float32)
                ss += jnp.dot(x_refs[i][...], q_refs[i][...], preferred_element_type=jnp.float32)
        xw_ref[...] = xw
        u_ref[...] = uu
        s_ref[...] = ss

    in_specs = (
        [pl.BlockSpec((_BN, xs[i].shape[1]), lambda i_: (i_, 0)) for i in range(nx)]
        + [pl.BlockSpec(Ws[i].shape, lambda i_: (0, 0)) for i in range(nw)]
        + [pl.BlockSpec(Us[i].shape, lambda i_: (0, 0)) for i in range(nw)]
        + [pl.BlockSpec(Qs[i].shape, lambda i_: (0, 0)) for i in range(nw)]
    )
    out_specs = [
        pl.BlockSpec((_BN, D9), lambda i_: (i_, 0)),
        pl.BlockSpec((_BN, DU), lambda i_: (i_, 0)),
        pl.BlockSpec((_BN, cout), lambda i_: (i_, 0)),
    ]
    out_shape = [
        jax.ShapeDtypeStruct((NP, D9), jnp.float32),
        jax.ShapeDtypeStruct((NP, DU), jnp.float32),
        jax.ShapeDtypeStruct((NP, cout), jnp.float32),
    ]
    return pl.pallas_call(
        body, grid=(grid,), in_specs=in_specs, out_specs=out_specs,
        out_shape=out_shape,
    )(*xs, *Ws[:nw], *Us[:nw], *Qs[:nw])


def _head_call(feat, w1, b1, w2, b2, xinit):
    NP = feat.shape[0]
    grid = pl.cdiv(NP, _BN)

    def body(f_ref, w1_ref, b1_ref, w2_ref, b2_ref, xi_ref, o_ref):
        h = jnp.dot(f_ref[...], w1_ref[...], preferred_element_type=jnp.float32)
        h = h + b1_ref[...]
        h = jnp.where(h > 0, h, 0.2 * h)
        o = jnp.dot(h, w2_ref[...], preferred_element_type=jnp.float32)
        o = o + b2_ref[...] + xi_ref[...]
        nrm = jnp.sqrt(jnp.sum(o * o, axis=1, keepdims=True))
        o_ref[...] = o / jnp.maximum(nrm, 1e-12)

    return pl.pallas_call(
        body,
        grid=(grid,),
        in_specs=[
            pl.BlockSpec((_BN, feat.shape[1]), lambda i: (i, 0)),
            pl.BlockSpec(w1.shape, lambda i: (0, 0)),
            pl.BlockSpec(b1.shape, lambda i: (0, 0)),
            pl.BlockSpec(w2.shape, lambda i: (0, 0)),
            pl.BlockSpec(b2.shape, lambda i: (0, 0)),
            pl.BlockSpec((_BN, 3), lambda i: (i, 0)),
        ],
        out_specs=pl.BlockSpec((_BN, 3), lambda i: (i, 0)),
        out_shape=jax.ShapeDtypeStruct((NP, 3), jnp.float32),
    )(feat, w1, b1, w2, b2, xinit)


@functools.lru_cache(maxsize=None)
def _sc_conv(cout, shift_g, shift_s, NPs, with_deg, EPT):
    W = -(-(cout + (16 if with_deg else 0)) // 128) * 128
    UOFF = _HEADS * cout
    D9 = -(-(UOFF + _HEADS) // 128) * 128
    NB = EPT // 16
    DEPTH = 2
    MBD = 2
    NK = cout // 16
    rps = NPs // 16
    mesh = plsc.VectorSubcoreMesh(core_axis_name="c", subcore_axis_name="s",
                                  num_cores=2, num_subcores=16)

    def body(xw_hbm, u_hbm, cvec_hbm, src_hbm, dst_hbm, zeros_hbm, out_hbm,
             src_v, dst_v, du_b, gx_b, mb_b, cv_v, agg_sp, du_s, gx_s, mb_s):
        cid = lax.axis_index("c")
        sid = lax.axis_index("s")
        wid = sid * 2 + cid
        pltpu.sync_copy(zeros_hbm.at[pl.ds(sid * rps, rps)],
                        agg_sp.at[pl.ds(sid * rps, rps)])
        pltpu.sync_copy(cvec_hbm, cv_v)
        plsc.subcore_barrier()
        cvec = cv_v[...]
        ii = lax.iota(jnp.int32, 16)
        degv = jnp.where(ii == 0, jnp.float32(1.0 if with_deg else 0.0),
                         jnp.float32(0.0))
        zv = jnp.zeros((16,), jnp.float32)
        for j in range(MBD):
            for e in range(16):
                for col in range(cout, W, 16):
                    mb_b[j][e, pl.ds(col, 16)] = degv if col == cout else zv

        sets = tuple((du_b[j], gx_b[j], du_s[j], gx_s[j],
                      mb_b[j % MBD], mb_s[j % MBD]) for j in range(DEPTH))

        def mk_idx(b):
            s_raw = src_v[pl.ds(b * 16, 16)]
            d_raw = dst_v[pl.ds(b * 16, 16)]
            sg = lax.shift_right_logical(s_raw, shift_g) if shift_g else s_raw
            dg = lax.shift_right_logical(d_raw, shift_g) if shift_g else d_raw
            dsc = lax.shift_right_logical(d_raw, shift_s) if shift_s else d_raw
            return sg, dg, dsc

        def issue(b, st):
            du, gx, dus, gxs = st[:4]
            sg, dg, _ = mk_idx(b)
            pltpu.async_copy(u_hbm.at[dg], du, dus)
            pltpu.async_copy(xw_hbm.at[sg], gx, gxs)

        def compute(b, st):
            du, gx, dus, gxs, mb, mbs = st
            sg, dg, dsc = mk_idx(b)
            pltpu.make_async_copy(u_hbm.at[dg], du, dus).wait()
            pltpu.make_async_copy(xw_hbm.at[sg], gx, gxs).wait()

            @pl.when(b >= MBD)
            def _():
                pltpu.make_async_copy(mb, agg_sp.at[dsc], mbs).wait()

            def edge_body(e, c2):
                t = gx[e, pl.ds(UOFF, 16)] - du[e, pl.ds(0, 16)] + cvec
                p = jnp.exp(t)
                ph = [p[h] for h in range(_HEADS)]
                ssum = ph[0]
                for h in range(1, _HEADS):
                    ssum = ssum + ph[h]
                invv = 1.0 / jnp.broadcast_to(ssum, (16,))
                accs = [jnp.zeros((16,), jnp.float32)] * NK
                for h in range(_HEADS):
                    for k in range(NK):
                        accs[k] = accs[k] + ph[h] * gx[e, pl.ds(h * cout + k * 16, 16)]
                for k in range(NK):
                    mb[e, pl.ds(k * 16, 16)] = accs[k] * invv
                return c2

            lax.fori_loop(0, 16, edge_body, 0)
            pltpu.async_copy(mb, agg_sp.at[dsc], mbs, add=True)

        zi = jnp.zeros((16,), jnp.int32)
        pltpu.sync_copy(src_hbm.at[pl.ds(wid * EPT, EPT)], src_v)
        pltpu.sync_copy(dst_hbm.at[pl.ds(wid * EPT, EPT)], dst_v)
        for j in range(DEPTH - 1):
            issue(j, sets[j])

        def quad_body(k, carry):
            b0 = DEPTH * k
            for j in range(DEPTH):
                b = b0 + j
                tgt = b + DEPTH - 1
                tset = sets[(j + DEPTH - 1) % DEPTH]
                if j == 0:
                    issue(tgt, tset)
                else:
                    @pl.when(tgt < NB)
                    def _():
                        issue(tgt, tset)

                compute(b, sets[j])
            return carry

        lax.fori_loop(0, NB // DEPTH, quad_body, 0)
        for j in range(MBD):
            pltpu.make_async_copy(mb_b[j], agg_sp.at[zi], mb_s[j]).wait()
        plsc.subcore_barrier()
        pltpu.sync_copy(agg_sp.at[pl.ds(sid * rps, rps)],
                        out_hbm.at[cid, pl.ds(sid * rps, rps)])

    return pl.kernel(
        body,
        out_type=jax.ShapeDtypeStruct((2, NPs, W), jnp.float32),
        mesh=mesh,
        scratch_types=[
            pltpu.VMEM((EPT,), jnp.int32),
            pltpu.VMEM((EPT,), jnp.int32),
            [pltpu.VMEM((16, 128), jnp.float32)] * DEPTH,
            [pltpu.VMEM((16, D9), jnp.float32)] * DEPTH,
            [pltpu.VMEM((16, W), jnp.float32)] * MBD,
            pltpu.VMEM((16,), jnp.float32),
            pltpu.VMEM_SHARED((NPs, W), jnp.float32),
            [pltpu.SemaphoreType.DMA] * DEPTH,
            [pltpu.SemaphoreType.DMA] * DEPTH,
            [pltpu.SemaphoreType.DMA] * MBD,
        ],
    )


def _split_params(p, splits):
    W, u, c, b = p["W"], p["u"], p["c"], p["b"]
    cout = b.shape[0]
    qc = jax.nn.softmax(c)
    wqc = (W.reshape(-1, _HEADS, cout) * qc[None, :, None]).sum(1)
    D9 = W.shape[1]
    D9p = -(-(D9 + _HEADS) // 128) * 128
    W = jnp.pad(jnp.concatenate([W, u], axis=1), ((0, 0), (0, D9p - D9 - _HEADS)))
    u16 = jnp.pad(u, ((0, 0), (0, 128 - _HEADS)))
    cvec = jnp.concatenate([c, jnp.full((16 - _HEADS,), _NEG, jnp.float32)])
    Ws, Us, Qs = [], [], []
    o = 0
    for s in splits:
        Ws.append(W[o:o + s])
        Us.append(u16[o:o + s])
        Qs.append(wqc[o:o + s])
        o += s
    return Ws, Us, Qs, cvec, b.reshape(1, cout)


def _lrelu(v):
    return jnp.where(v > 0, v, 0.2 * v)


def kernel(x, edge_index, params):
    n1 = x.shape[0]
    n2 = (n1 + 1) // 2
    n3 = (n2 + 1) // 2
    NP1, NP2, NP3 = [-(-(n + 1) // 128) * 128 for n in (n1, n2, n3)]
    E = edge_index.shape[1]
    EPT = -(-(-(-E // 32)) // 64) * 64
    EP = 32 * EPT
    src = jnp.concatenate([edge_index[0], jnp.full((EP - E,), n1, jnp.int32)])
    dst = jnp.concatenate([edge_index[1], jnp.full((EP - E,), n1, jnp.int32)])
    f32 = jnp.float32

    def conv(xs, pname, splits, pool, shift_g, shift_s, NPg, NPs, with_deg=False):
        Ws, Us, Qs, cvec, b = _split_params(params[pname], splits)
        cout = b.shape[1]
        xw, u16t, s = _prep_call(xs, Ws, Us, Qs, pool)
        Wagg = -(-(cout + (16 if with_deg else 0)) // 128) * 128
        zeros = jnp.zeros((NPs, Wagg), f32)
        part = _sc_conv(cout, shift_g, shift_s, NPs, with_deg, EPT)(
            xw, u16t, cvec, src, dst, zeros)
        return part, s, b

    xp = jnp.pad(x, ((0, NP1 - n1), (0, 0)))
    p1, s1, b1 = conv([xp], "l1", [6], False, 0, 0, NP1, NP1, with_deg=True)
    indeg1 = p1[0, :, 32] + p1[1, :, 32]
    deg1 = indeg1 + 1.0
    deg2 = jnp.pad(indeg1.reshape(NP1 // 2, 2).sum(1),
                   (0, NP2 - NP1 // 2)) + 1.0
    deg3 = jnp.pad((deg2 - 1.0)[:NP2].reshape(NP2 // 2, 2).sum(1),
                   (0, NP3 - NP2 // 2)) + 1.0
    y1 = _lrelu((p1[0, :, :32] + p1[1, :, :32] + s1) / deg1[:, None] + b1)

    pad2 = ((0, NP2 - NP1 // 2), (0, 0))
    p2, s2, b2 = conv([jnp.pad(y1[0::2], pad2), jnp.pad(y1[1::2], pad2)],
                      "l2", [32], True, 1, 1, NP2, NP2)
    y2 = _lrelu((p2[0, :, :64] + p2[1, :, :64] + s2) / deg2[:, None] + b2)

    pad3 = ((0, NP3 - NP2 // 2), (0, 0))
    p3, s3, b3 = conv([jnp.pad(y2[0::2], pad3), jnp.pad(y2[1::2], pad3)],
                      "l3", [64], True, 2, 2, NP3, NP3)
    y3a = _lrelu((p3[0, :, :128] + p3[1, :, :128] + s3) / deg3[:, None] + b3)

    p4, s4, b4 = conv([y3a], "l4", [128], False, 2, 2, NP3, NP3)
    y3 = _lrelu((p4[0, :, :128] + p4[1, :, :128] + s4) / deg3[:, None] + b4)

    pr1, sr1, br1 = conv([y3], "r1", [128], False, 2, 1, NP3, NP2)
    sr1 = jnp.repeat(sr1[:NP2 // 2], 2, axis=0)
    f2p = (pr1[0, :, :64] + pr1[1, :, :64] + sr1) / deg2[:, None] + br1

    pr2, sr2, br2 = conv([y2, f2p], "r2", [64, 64], False, 1, 1, NP2, NP2)
    y2c = _lrelu((pr2[0, :, :64] + pr2[1, :, :64] + sr2) / deg2[:, None] + br2)

    pr3, sr3, br3 = conv([y2c], "r3", [64], False, 1, 0, NP2, NP1)
    sr3 = jnp.repeat(sr3[:NP1 // 2], 2, axis=0)
    f1p = (pr3[0, :, :32] + pr3[1, :, :32] + sr3) / deg1[:, None] + br3

    pr4, sr4, br4 = conv([y1, f1p], "r4", [32, 32], False, 0, 0, NP1, NP1)
    feat = _lrelu((pr4[0, :, :32] + pr4[1, :, :32] + sr4) / deg1[:, None] + br4)

    xinit = jnp.pad(x[:, 3:6], ((0, NP1 - n1), (0, 0)))
    out = _head_call(feat, params["fc_v1"]["W"], params["fc_v1"]["b"].reshape(1, -1),
                     params["fc_v2"]["W"], params["fc_v2"]["b"].reshape(1, -1), xinit)
    return out[:n1]


# speedup vs baseline: 1.1231x; 1.1229x over previous
import functools

import jax
import jax.numpy as jnp
from jax import lax
from jax.experimental import pallas as pl
from jax.experimental.pallas import tpu as pltpu
from jax.experimental.pallas import tpu_sc as plsc

_HEADS = 9
_NEG = -1e30
_BN = 1024


def _prep_call(xs, Ws, Us, Qs, pool):
    NP = xs[0].shape[0]
    nx = len(xs)
    D9 = Ws[0].shape[1]
    DU = Us[0].shape[1]
    cout = Qs[0].shape[1]
    grid = pl.cdiv(NP, _BN)

    nw = 1 if pool else nx

    def body(*refs):
        x_refs = refs[:nx]
        w_refs = refs[nx:nx + nw]
        u_refs = refs[nx + nw:nx + 2 * nw]
        q_refs = refs[nx + 2 * nw:nx + 3 * nw]
        xw_ref, u_ref, s_ref = refs[nx + 3 * nw:]
        if pool:
            xin = jnp.maximum(x_refs[0][...], x_refs[1][...])
            xw = jnp.dot(xin, w_refs[0][...], preferred_element_type=jnp.float32)
            uu = jnp.dot(xin, u_refs[0][...], preferred_element_type=jnp.float32)
            ss = jnp.dot(xin, q_refs[0][...], preferred_element_type=jnp.float32)
        else:
            xw = jnp.dot(x_refs[0][...], w_refs[0][...], preferred_element_type=jnp.float32)
            uu = jnp.dot(x_refs[0][...], u_refs[0][...], preferred_element_type=jnp.float32)
            ss = jnp.dot(x_refs[0][...], q_refs[0][...], preferred_element_type=jnp.float32)
            for i in range(1, nx):
                xw += jnp.dot(x_refs[i][...], w_refs[i][...], preferred_element_type=jnp.float32)
                uu += jnp.dot(x_refs[i][...], u_refs[i][...], preferred_element_type=jnp.float32)
                ss += jnp.dot(x_refs[i][...], q_refs[i][...], preferred_element_type=jnp.float32)
        xw_ref[...] = xw
        u_ref[...] = uu
        s_ref[...] = ss

    in_specs = (
        [pl.BlockSpec((_BN, xs[i].shape[1]), lambda i_: (i_, 0)) for i in range(nx)]
        + [pl.BlockSpec(Ws[i].shape, lambda i_: (0, 0)) for i in range(nw)]
        + [pl.BlockSpec(Us[i].shape, lambda i_: (0, 0)) for i in range(nw)]
        + [pl.BlockSpec(Qs[i].shape, lambda i_: (0, 0)) for i in range(nw)]
    )
    out_specs = [
        pl.BlockSpec((_BN, D9), lambda i_: (i_, 0)),
        pl.BlockSpec((_BN, DU), lambda i_: (i_, 0)),
        pl.BlockSpec((_BN, cout), lambda i_: (i_, 0)),
    ]
    out_shape = [
        jax.ShapeDtypeStruct((NP, D9), jnp.float32),
        jax.ShapeDtypeStruct((NP, DU), jnp.float32),
        jax.ShapeDtypeStruct((NP, cout), jnp.float32),
    ]
    return pl.pallas_call(
        body, grid=(grid,), in_specs=in_specs, out_specs=out_specs,
        out_shape=out_shape,
    )(*xs, *Ws[:nw], *Us[:nw], *Qs[:nw])


def _head_call(feat, w1, b1, w2, b2, xinit):
    NP = feat.shape[0]
    grid = pl.cdiv(NP, _BN)

    def body(f_ref, w1_ref, b1_ref, w2_ref, b2_ref, xi_ref, o_ref):
        h = jnp.dot(f_ref[...], w1_ref[...], preferred_element_type=jnp.float32)
        h = h + b1_ref[...]
        h = jnp.where(h > 0, h, 0.2 * h)
        o = jnp.dot(h, w2_ref[...], preferred_element_type=jnp.float32)
        o = o + b2_ref[...] + xi_ref[...]
        nrm = jnp.sqrt(jnp.sum(o * o, axis=1, keepdims=True))
        o_ref[...] = o / jnp.maximum(nrm, 1e-12)

    return pl.pallas_call(
        body,
        grid=(grid,),
        in_specs=[
            pl.BlockSpec((_BN, feat.shape[1]), lambda i: (i, 0)),
            pl.BlockSpec(w1.shape, lambda i: (0, 0)),
            pl.BlockSpec(b1.shape, lambda i: (0, 0)),
            pl.BlockSpec(w2.shape, lambda i: (0, 0)),
            pl.BlockSpec(b2.shape, lambda i: (0, 0)),
            pl.BlockSpec((_BN, 3), lambda i: (i, 0)),
        ],
        out_specs=pl.BlockSpec((_BN, 3), lambda i: (i, 0)),
        out_shape=jax.ShapeDtypeStruct((NP, 3), jnp.float32),
    )(feat, w1, b1, w2, b2, xinit)


@functools.lru_cache(maxsize=None)
def _sc_conv(cout, shift_g, shift_s, NPs, with_deg, EPT):
    W = -(-(cout + (16 if with_deg else 0)) // 128) * 128
    UOFF = _HEADS * cout
    D9 = -(-(UOFF + _HEADS) // 128) * 128
    NB = EPT // 16
    DEPTH = 2
    MBD = 2
    NK = cout // 16
    rps = NPs // 16
    mesh = plsc.VectorSubcoreMesh(core_axis_name="c", subcore_axis_name="s",
                                  num_cores=2, num_subcores=16)

    def body(xw_hbm, u_hbm, cvec_hbm, src_hbm, dst_hbm, zeros_hbm, out_hbm,
             src_v, dst_v, du_b, gx_b, mb_b, cv_v, agg_sp, du_s, gx_s, mb_s):
        cid = lax.axis_index("c")
        sid = lax.axis_index("s")
        wid = sid * 2 + cid
        pltpu.sync_copy(zeros_hbm.at[pl.ds(sid * rps, rps)],
                        agg_sp.at[pl.ds(sid * rps, rps)])
        pltpu.sync_copy(cvec_hbm, cv_v)
        plsc.subcore_barrier()
        cvec = cv_v[...]
        ii = lax.iota(jnp.int32, 16)
        degv = jnp.where(ii == 0, jnp.float32(1.0 if with_deg else 0.0),
                         jnp.float32(0.0))
        zv = jnp.zeros((16,), jnp.float32)
        for j in range(MBD):
            for e in range(16):
                for col in range(cout, W, 16):
                    mb_b[j][e, pl.ds(col, 16)] = degv if col == cout else zv

        sets = tuple((du_b[j], gx_b[j], du_s[j], gx_s[j],
                      mb_b[j % MBD], mb_s[j % MBD]) for j in range(DEPTH))

        def mk_idx(b):
            s_raw = src_v[pl.ds(b * 16, 16)]
            d_raw = dst_v[pl.ds(b * 16, 16)]
            sg = lax.shift_right_logical(s_raw, shift_g) if shift_g else s_raw
            dg = lax.shift_right_logical(d_raw, shift_g) if shift_g else d_raw
            dsc = lax.shift_right_logical(d_raw, shift_s) if shift_s else d_raw
            return sg, dg, dsc

        def issue(b, st):
            du, gx, dus, gxs = st[:4]
            sg, dg, _ = mk_idx(b)
            pltpu.async_copy(u_hbm.at[dg], du, dus)
            pltpu.async_copy(xw_hbm.at[sg], gx, gxs)

        def compute(b, st):
            du, gx, dus, gxs, mb, mbs = st
            sg, dg, dsc = mk_idx(b)
            pltpu.make_async_copy(u_hbm.at[dg], du, dus).wait()
            pltpu.make_async_copy(xw_hbm.at[sg], gx, gxs).wait()

            @pl.when(b >= MBD)
            def _():
                pltpu.make_async_copy(mb, agg_sp.at[dsc], mbs).wait()

            def edge_body(e, c2):
                t = gx[e, pl.ds(UOFF, 16)] - du[e, pl.ds(0, 16)] + cvec
                p = jnp.exp(t)
                ph = [p[h] for h in range(_HEADS)]
                ssum = ph[0]
                for h in range(1, _HEADS):
                    ssum = ssum + ph[h]
                invv = 1.0 / jnp.broadcast_to(ssum, (16,))
                accs = [jnp.zeros((16,), jnp.float32)] * NK
                for h in range(_HEADS):
                    for k in range(NK):
                        accs[k] = accs[k] + ph[h] * gx[e, pl.ds(h * cout + k * 16, 16)]
                for k in range(NK):
                    mb[e, pl.ds(k * 16, 16)] = accs[k] * invv
                return c2

            lax.fori_loop(0, 16, edge_body, 0)
            pltpu.async_copy(mb, agg_sp.at[dsc], mbs, add=True)

        zi = jnp.zeros((16,), jnp.int32)
        pltpu.sync_copy(src_hbm.at[pl.ds(wid * EPT, EPT)], src_v)
        pltpu.sync_copy(dst_hbm.at[pl.ds(wid * EPT, EPT)], dst_v)
        for j in range(DEPTH - 1):
            issue(j, sets[j])

        def quad_body(k, carry):
            b0 = DEPTH * k
            for j in range(DEPTH):
                b = b0 + j
                tgt = b + DEPTH - 1
                tset = sets[(j + DEPTH - 1) % DEPTH]
                if j == 0:
                    issue(tgt, tset)
                else:
                    @pl.when(tgt < NB)
                    def _():
                        issue(tgt, tset)

                compute(b, sets[j])
            return carry

        lax.fori_loop(0, NB // DEPTH, quad_body, 0)
        for j in range(MBD):
            pltpu.make_async_copy(mb_b[j], agg_sp.at[zi], mb_s[j]).wait()
        plsc.subcore_barrier()
        pltpu.sync_copy(agg_sp.at[pl.ds(sid * rps, rps)],
                        out_hbm.at[cid, pl.ds(sid * rps, rps)])

    return pl.kernel(
        body,
        out_type=jax.ShapeDtypeStruct((2, NPs, W), jnp.float32),
        mesh=mesh,
        scratch_types=[
            pltpu.VMEM((EPT,), jnp.int32),
            pltpu.VMEM((EPT,), jnp.int32),
            [pltpu.VMEM((16, 128), jnp.float32)] * DEPTH,
            [pltpu.VMEM((16, D9), jnp.float32)] * DEPTH,
            [pltpu.VMEM((16, W), jnp.float32)] * MBD,
            pltpu.VMEM((16,), jnp.float32),
            pltpu.VMEM_SHARED((NPs, W), jnp.float32),
            [pltpu.SemaphoreType.DMA] * DEPTH,
            [pltpu.SemaphoreType.DMA] * DEPTH,
            [pltpu.SemaphoreType.DMA] * MBD,
        ],
    )


def _split_params(p, splits):
    W, u, c, b = p["W"], p["u"], p["c"], p["b"]
    cout = b.shape[0]
    qc = jax.nn.softmax(c)
    wqc = (W.reshape(-1, _HEADS, cout) * qc[None, :, None]).sum(1)
    D9 = W.shape[1]
    D9p = -(-(D9 + _HEADS) // 128) * 128
    W = jnp.pad(jnp.concatenate([W, u], axis=1), ((0, 0), (0, D9p - D9 - _HEADS)))
    u16 = jnp.pad(u, ((0, 0), (0, 128 - _HEADS)))
    cvec = jnp.concatenate([c, jnp.full((16 - _HEADS,), _NEG, jnp.float32)])
    Ws, Us, Qs = [], [], []
    o = 0
    for s in splits:
        Ws.append(W[o:o + s])
        Us.append(u16[o:o + s])
        Qs.append(wqc[o:o + s])
        o += s
    return Ws, Us, Qs, cvec, b.reshape(1, cout)


def _lrelu(v):
    return jnp.where(v > 0, v, 0.2 * v)


def kernel(x, edge_index, params):
    n1 = x.shape[0]
    n2 = (n1 + 1) // 2
    n3 = (n2 + 1) // 2
    NP1, NP2, NP3 = [-(-(n + 1) // 128) * 128 for n in (n1, n2, n3)]
    E = edge_index.shape[1]
    EPT = -(-(-(-E // 32)) // 32) * 32
    EP = 32 * EPT
    src = jnp.concatenate([edge_index[0], jnp.full((EP - E,), n1, jnp.int32)])
    dst = jnp.concatenate([edge_index[1], jnp.full((EP - E,), n1, jnp.int32)])
    f32 = jnp.float32

    def conv(xs, pname, splits, pool, shift_g, shift_s, NPg, NPs, with_deg=False):
        Ws, Us, Qs, cvec, b = _split_params(params[pname], splits)
        cout = b.shape[1]
        xw, u16t, s = _prep_call(xs, Ws, Us, Qs, pool)
        Wagg = -(-(cout + (16 if with_deg else 0)) // 128) * 128
        zeros = jnp.zeros((NPs, Wagg), f32)
        part = _sc_conv(cout, shift_g, shift_s, NPs, with_deg, EPT)(
            xw, u16t, cvec, src, dst, zeros)
        return part, s, b

    xp = jnp.pad(x, ((0, NP1 - n1), (0, 0)))
    p1, s1, b1 = conv([xp], "l1", [6], False, 0, 0, NP1, NP1, with_deg=True)
    indeg1 = p1[0, :, 32] + p1[1, :, 32]
    deg1 = indeg1 + 1.0
    deg2 = jnp.pad(indeg1.reshape(NP1 // 2, 2).sum(1),
                   (0, NP2 - NP1 // 2)) + 1.0
    deg3 = jnp.pad((deg2 - 1.0)[:NP2].reshape(NP2 // 2, 2).sum(1),
                   (0, NP3 - NP2 // 2)) + 1.0
    y1 = _lrelu((p1[0, :, :32] + p1[1, :, :32] + s1) / deg1[:, None] + b1)

    pad2 = ((0, NP2 - NP1 // 2), (0, 0))
    p2, s2, b2 = conv([jnp.pad(y1[0::2], pad2), jnp.pad(y1[1::2], pad2)],
                      "l2", [32], True, 1, 1, NP2, NP2)
    y2 = _lrelu((p2[0, :, :64] + p2[1, :, :64] + s2) / deg2[:, None] + b2)

    pad3 = ((0, NP3 - NP2 // 2), (0, 0))
    p3, s3, b3 = conv([jnp.pad(y2[0::2], pad3), jnp.pad(y2[1::2], pad3)],
                      "l3", [64], True, 2, 2, NP3, NP3)
    y3a = _lrelu((p3[0, :, :128] + p3[1, :, :128] + s3) / deg3[:, None] + b3)

    p4, s4, b4 = conv([y3a], "l4", [128], False, 2, 2, NP3, NP3)
    y3 = _lrelu((p4[0, :, :128] + p4[1, :, :128] + s4) / deg3[:, None] + b4)

    pr1, sr1, br1 = conv([y3], "r1", [128], False, 2, 1, NP3, NP2)
    sr1 = jnp.repeat(sr1[:NP2 // 2], 2, axis=0)
    f2p = (pr1[0, :, :64] + pr1[1, :, :64] + sr1) / deg2[:, None] + br1

    pr2, sr2, br2 = conv([y2, f2p], "r2", [64, 64], False, 1, 1, NP2, NP2)
    y2c = _lrelu((pr2[0, :, :64] + pr2[1, :, :64] + sr2) / deg2[:, None] + br2)

    pr3, sr3, br3 = conv([y2c], "r3", [64], False, 1, 0, NP2, NP1)
    sr3 = jnp.repeat(sr3[:NP1 // 2], 2, axis=0)
    f1p = (pr3[0, :, :32] + pr3[1, :, :32] + sr3) / deg1[:, None] + br3

    pr4, sr4, br4 = conv([y1, f1p], "r4", [32, 32], False, 0, 0, NP1, NP1)
    feat = _lrelu((pr4[0, :, :32] + pr4[1, :, :32] + sr4) / deg1[:, None] + br4)

    xinit = jnp.pad(x[:, 3:6], ((0, NP1 - n1), (0, 0)))
    out = _head_call(feat, params["fc_v1"]["W"], params["fc_v1"]["b"].reshape(1, -1),
                     params["fc_v2"]["W"], params["fc_v2"]["b"].reshape(1, -1), xinit)
    return out[:n1]


# speedup vs baseline: 1.1728x; 1.0442x over previous
import functools

import jax
import jax.numpy as jnp
from jax import lax
from jax.experimental import pallas as pl
from jax.experimental.pallas import tpu as pltpu
from jax.experimental.pallas import tpu_sc as plsc

_HEADS = 9
_NEG = -1e30
_BN = 1024


def _prep_call(xs, Ws, Us, Qs, pool):
    NP = xs[0].shape[0]
    nx = len(xs)
    D9 = Ws[0].shape[1]
    DU = Us[0].shape[1]
    cout = Qs[0].shape[1]
    grid = pl.cdiv(NP, _BN)

    nw = 1 if pool else nx

    def body(*refs):
        x_refs = refs[:nx]
        w_refs = refs[nx:nx + nw]
        u_refs = refs[nx + nw:nx + 2 * nw]
        q_refs = refs[nx + 2 * nw:nx + 3 * nw]
        xw_ref, u_ref, s_ref = refs[nx + 3 * nw:]
        if pool:
            xin = jnp.maximum(x_refs[0][...], x_refs[1][...])
            xw = jnp.dot(xin, w_refs[0][...], preferred_element_type=jnp.float32)
            uu = jnp.dot(xin, u_refs[0][...], preferred_element_type=jnp.float32)
            ss = jnp.dot(xin, q_refs[0][...], preferred_element_type=jnp.float32)
        else:
            xw = jnp.dot(x_refs[0][...], w_refs[0][...], preferred_element_type=jnp.float32)
            uu = jnp.dot(x_refs[0][...], u_refs[0][...], preferred_element_type=jnp.float32)
            ss = jnp.dot(x_refs[0][...], q_refs[0][...], preferred_element_type=jnp.float32)
            for i in range(1, nx):
                xw += jnp.dot(x_refs[i][...], w_refs[i][...], preferred_element_type=jnp.float32)
                uu += jnp.dot(x_refs[i][...], u_refs[i][...], preferred_element_type=jnp.float32)
                ss += jnp.dot(x_refs[i][...], q_refs[i][...], preferred_element_type=jnp.float32)
        xw_ref[...] = xw
        u_ref[...] = uu
        s_ref[...] = ss

    in_specs = (
        [pl.BlockSpec((_BN, xs[i].shape[1]), lambda i_: (i_, 0)) for i in range(nx)]
        + [pl.BlockSpec(Ws[i].shape, lambda i_: (0, 0)) for i in range(nw)]
        + [pl.BlockSpec(Us[i].shape, lambda i_: (0, 0)) for i in range(nw)]
        + [pl.BlockSpec(Qs[i].shape, lambda i_: (0, 0)) for i in range(nw)]
    )
    out_specs = [
        pl.BlockSpec((_BN, D9), lambda i_: (i_, 0)),
        pl.BlockSpec((_BN, DU), lambda i_: (i_, 0)),
        pl.BlockSpec((_BN, cout), lambda i_: (i_, 0)),
    ]
    out_shape = [
        jax.ShapeDtypeStruct((NP, D9), jnp.float32),
        jax.ShapeDtypeStruct((NP, DU), jnp.float32),
        jax.ShapeDtypeStruct((NP, cout), jnp.float32),
    ]
    return pl.pallas_call(
        body, grid=(grid,), in_specs=in_specs, out_specs=out_specs,
        out_shape=out_shape,
    )(*xs, *Ws[:nw], *Us[:nw], *Qs[:nw])


def _head_call(feat, w1, b1, w2, b2, xinit):
    NP = feat.shape[0]
    grid = pl.cdiv(NP, _BN)

    def body(f_ref, w1_ref, b1_ref, w2_ref, b2_ref, xi_ref, o_ref):
        h = jnp.dot(f_ref[...], w1_ref[...], preferred_element_type=jnp.float32)
        h = h + b1_ref[...]
        h = jnp.where(h > 0, h, 0.2 * h)
        o = jnp.dot(h, w2_ref[...], preferred_element_type=jnp.float32)
        o = o + b2_ref[...] + xi_ref[...]
        nrm = jnp.sqrt(jnp.sum(o * o, axis=1, keepdims=True))
        o_ref[...] = o / jnp.maximum(nrm, 1e-12)

    return pl.pallas_call(
        body,
        grid=(grid,),
        in_specs=[
            pl.BlockSpec((_BN, feat.shape[1]), lambda i: (i, 0)),
            pl.BlockSpec(w1.shape, lambda i: (0, 0)),
            pl.BlockSpec(b1.shape, lambda i: (0, 0)),
            pl.BlockSpec(w2.shape, lambda i: (0, 0)),
            pl.BlockSpec(b2.shape, lambda i: (0, 0)),
            pl.BlockSpec((_BN, 3), lambda i: (i, 0)),
        ],
        out_specs=pl.BlockSpec((_BN, 3), lambda i: (i, 0)),
        out_shape=jax.ShapeDtypeStruct((NP, 3), jnp.float32),
    )(feat, w1, b1, w2, b2, xinit)


@functools.lru_cache(maxsize=None)
def _sc_conv(cout, shift_g, shift_s, NPs, with_deg, EPT):
    W = -(-(cout + (16 if with_deg else 0)) // 128) * 128
    UOFF = _HEADS * cout
    D9 = -(-(UOFF + _HEADS) // 128) * 128
    NB = EPT // 16
    DEPTH = 2
    MBD = 2
    NK = cout // 16
    rps = NPs // 16
    mesh = plsc.VectorSubcoreMesh(core_axis_name="c", subcore_axis_name="s",
                                  num_cores=2, num_subcores=16)

    def body(xw_hbm, u_hbm, cvec_hbm, src_hbm, dst_hbm, zeros_hbm, out_hbm,
             src_v, dst_v, du_b, gx_b, mb_b, cv_v, agg_sp, du_s, gx_s, mb_s):
        cid = lax.axis_index("c")
        sid = lax.axis_index("s")
        wid = sid * 2 + cid
        pltpu.sync_copy(zeros_hbm.at[pl.ds(sid * rps, rps)],
                        agg_sp.at[pl.ds(sid * rps, rps)])
        pltpu.sync_copy(cvec_hbm, cv_v)
        plsc.subcore_barrier()
        cvec = cv_v[...]
        ii = lax.iota(jnp.int32, 16)
        degv = jnp.where(ii == 0, jnp.float32(1.0 if with_deg else 0.0),
                         jnp.float32(0.0))
        zv = jnp.zeros((16,), jnp.float32)
        for j in range(MBD):
            for e in range(16):
                for col in range(cout, W, 16):
                    mb_b[j][e, pl.ds(col, 16)] = degv if col == cout else zv

        sets = tuple((du_b[j], gx_b[j], du_s[j], gx_s[j],
                      mb_b[j % MBD], mb_s[j % MBD]) for j in range(DEPTH))

        def mk_idx(b):
            s_raw = src_v[pl.ds(b * 16, 16)]
            d_raw = dst_v[pl.ds(b * 16, 16)]
            sg = lax.shift_right_logical(s_raw, shift_g) if shift_g else s_raw
            dg = lax.shift_right_logical(d_raw, shift_g) if shift_g else d_raw
            dsc = lax.shift_right_logical(d_raw, shift_s) if shift_s else d_raw
            return sg, dg, dsc

        def issue(b, st):
            du, gx, dus, gxs = st[:4]
            sg, dg, _ = mk_idx(b)
            pltpu.async_copy(u_hbm.at[dg], du, dus)
            pltpu.async_copy(xw_hbm.at[sg], gx, gxs)

        def compute(b, st):
            du, gx, dus, gxs, mb, mbs = st
            sg, dg, dsc = mk_idx(b)
            pltpu.make_async_copy(u_hbm.at[dg], du, dus).wait()
            pltpu.make_async_copy(xw_hbm.at[sg], gx, gxs).wait()

            @pl.when(b >= MBD)
            def _():
                pltpu.make_async_copy(mb, agg_sp.at[dsc], mbs).wait()

            def edge_body(e, c2):
                t = gx[e, pl.ds(UOFF, 16)] - du[e, pl.ds(0, 16)] + cvec
                p = jnp.exp(t)
                ph = [p[h] for h in range(_HEADS)]
                ssum = ph[0]
                for h in range(1, _HEADS):
                    ssum = ssum + ph[h]
                invv = 1.0 / jnp.broadcast_to(ssum, (16,))
                accs = [jnp.zeros((16,), jnp.float32)] * NK
                for h in range(_HEADS):
                    for k in range(NK):
                        accs[k] = accs[k] + ph[h] * gx[e, pl.ds(h * cout + k * 16, 16)]
                for k in range(NK):
                    mb[e, pl.ds(k * 16, 16)] = accs[k] * invv
                return c2

            lax.fori_loop(0, 16, edge_body, 0)
            pltpu.async_copy(mb, agg_sp.at[dsc], mbs, add=True)

        zi = jnp.zeros((16,), jnp.int32)
        pltpu.sync_copy(src_hbm.at[pl.ds(wid * EPT, EPT)], src_v)
        pltpu.sync_copy(dst_hbm.at[pl.ds(wid * EPT, EPT)], dst_v)
        for j in range(DEPTH - 1):
            issue(j, sets[j])

        def quad_body(k, carry):
            b0 = DEPTH * k
            for j in range(DEPTH):
                b = b0 + j
                tgt = b + DEPTH - 1
                tset = sets[(j + DEPTH - 1) % DEPTH]
                if j == 0:
                    issue(tgt, tset)
                else:
                    @pl.when(tgt < NB)
                    def _():
                        issue(tgt, tset)

                compute(b, sets[j])
            return carry

        lax.fori_loop(0, NB // DEPTH, quad_body, 0)
        for j in range(MBD):
            pltpu.make_async_copy(mb_b[j], agg_sp.at[zi], mb_s[j]).wait()
        plsc.subcore_barrier()
        pltpu.sync_copy(agg_sp.at[pl.ds(sid * rps, rps)],
                        out_hbm.at[cid, pl.ds(sid * rps, rps)])

    return pl.kernel(
        body,
        out_type=jax.ShapeDtypeStruct((2, NPs, W), jnp.float32),
        mesh=mesh,
        scratch_types=[
            pltpu.VMEM((EPT,), jnp.int32),
            pltpu.VMEM((EPT,), jnp.int32),
            [pltpu.VMEM((16, 128), jnp.float32)] * DEPTH,
            [pltpu.VMEM((16, D9), jnp.float32)] * DEPTH,
            [pltpu.VMEM((16, W), jnp.float32)] * MBD,
            pltpu.VMEM((16,), jnp.float32),
            pltpu.VMEM_SHARED((NPs, W), jnp.float32),
            [pltpu.SemaphoreType.DMA] * DEPTH,
            [pltpu.SemaphoreType.DMA] * DEPTH,
            [pltpu.SemaphoreType.DMA] * MBD,
        ],
    )


def _split_params(p, splits):
    W, u, c, b = p["W"], p["u"], p["c"], p["b"]
    cout = b.shape[0]
    qc = jax.nn.softmax(c)
    wqc = (W.reshape(-1, _HEADS, cout) * qc[None, :, None]).sum(1)
    D9 = W.shape[1]
    D9p = -(-(D9 + _HEADS) // 128) * 128
    W = jnp.pad(jnp.concatenate([W, u], axis=1), ((0, 0), (0, D9p - D9 - _HEADS)))
    u16 = jnp.pad(u, ((0, 0), (0, 128 - _HEADS)))
    cvec = jnp.concatenate([c, jnp.full((16 - _HEADS,), _NEG, jnp.float32)])
    Ws, Us, Qs = [], [], []
    o = 0
    for s in splits:
        Ws.append(W[o:o + s])
        Us.append(u16[o:o + s])
        Qs.append(wqc[o:o + s])
        o += s
    return Ws, Us, Qs, cvec, b.reshape(1, cout)


def _lrelu(v):
    return jnp.where(v > 0, v, 0.2 * v)


def kernel(x, edge_index, params):
    n1 = x.shape[0]
    n2 = (n1 + 1) // 2
    n3 = (n2 + 1) // 2
    NP1, NP2, NP3 = [-(-(n + 1) // 128) * 128 for n in (n1, n2, n3)]
    E = edge_index.shape[1]
    EPT = -(-(-(-E // 32)) // 32) * 32
    EP = 32 * EPT
    padv = n1 + jnp.arange(EP - E, dtype=jnp.int32) % (NP1 - n1)
    src = jnp.concatenate([edge_index[0], padv])
    dst = jnp.concatenate([edge_index[1], padv])
    f32 = jnp.float32

    def conv(xs, pname, splits, pool, shift_g, shift_s, NPg, NPs, with_deg=False):
        Ws, Us, Qs, cvec, b = _split_params(params[pname], splits)
        cout = b.shape[1]
        xw, u16t, s = _prep_call(xs, Ws, Us, Qs, pool)
        Wagg = -(-(cout + (16 if with_deg else 0)) // 128) * 128
        zeros = jnp.zeros((NPs, Wagg), f32)
        part = _sc_conv(cout, shift_g, shift_s, NPs, with_deg, EPT)(
            xw, u16t, cvec, src, dst, zeros)
        return part, s, b

    xp = jnp.pad(x, ((0, NP1 - n1), (0, 0)))
    p1, s1, b1 = conv([xp], "l1", [6], False, 0, 0, NP1, NP1, with_deg=True)
    indeg1 = p1[0, :, 32] + p1[1, :, 32]
    deg1 = indeg1 + 1.0
    deg2 = jnp.pad(indeg1.reshape(NP1 // 2, 2).sum(1),
                   (0, NP2 - NP1 // 2)) + 1.0
    deg3 = jnp.pad((deg2 - 1.0)[:NP2].reshape(NP2 // 2, 2).sum(1),
                   (0, NP3 - NP2 // 2)) + 1.0
    y1 = _lrelu((p1[0, :, :32] + p1[1, :, :32] + s1) / deg1[:, None] + b1)

    pad2 = ((0, NP2 - NP1 // 2), (0, 0))
    p2, s2, b2 = conv([jnp.pad(y1[0::2], pad2), jnp.pad(y1[1::2], pad2)],
                      "l2", [32], True, 1, 1, NP2, NP2)
    y2 = _lrelu((p2[0, :, :64] + p2[1, :, :64] + s2) / deg2[:, None] + b2)

    pad3 = ((0, NP3 - NP2 // 2), (0, 0))
    p3, s3, b3 = conv([jnp.pad(y2[0::2], pad3), jnp.pad(y2[1::2], pad3)],
                      "l3", [64], True, 2, 2, NP3, NP3)
    y3a = _lrelu((p3[0, :, :128] + p3[1, :, :128] + s3) / deg3[:, None] + b3)

    p4, s4, b4 = conv([y3a], "l4", [128], False, 2, 2, NP3, NP3)
    y3 = _lrelu((p4[0, :, :128] + p4[1, :, :128] + s4) / deg3[:, None] + b4)

    pr1, sr1, br1 = conv([y3], "r1", [128], False, 2, 1, NP3, NP2)
    sr1 = jnp.repeat(sr1[:NP2 // 2], 2, axis=0)
    f2p = (pr1[0, :, :64] + pr1[1, :, :64] + sr1) / deg2[:, None] + br1

    pr2, sr2, br2 = conv([y2, f2p], "r2", [64, 64], False, 1, 1, NP2, NP2)
    y2c = _lrelu((pr2[0, :, :64] + pr2[1, :, :64] + sr2) / deg2[:, None] + br2)

    pr3, sr3, br3 = conv([y2c], "r3", [64], False, 1, 0, NP2, NP1)
    sr3 = jnp.repeat(sr3[:NP1 // 2], 2, axis=0)
    f1p = (pr3[0, :, :32] + pr3[1, :, :32] + sr3) / deg1[:, None] + br3

    pr4, sr4, br4 = conv([y1, f1p], "r4", [32, 32], False, 0, 0, NP1, NP1)
    feat = _lrelu((pr4[0, :, :32] + pr4[1, :, :32] + sr4) / deg1[:, None] + br4)

    xinit = jnp.pad(x[:, 3:6], ((0, NP1 - n1), (0, 0)))
    out = _head_call(feat, params["fc_v1"]["W"], params["fc_v1"]["b"].reshape(1, -1),
                     params["fc_v2"]["W"], params["fc_v2"]["b"].reshape(1, -1), xinit)
    return out[:n1]
